# baseline (speedup 1.0000x reference)
import functools

import jax
import jax.numpy as jnp
from jax import lax
from jax.experimental import pallas as pl
from jax.experimental.pallas import tpu as pltpu
from jax.experimental.pallas import tpu_sc as plsc

N_NODES = 10000
N_PAD = 10240
E_EDGES = 320000
D_IN = 128
H_HID = 128
C_OUT = 64

NC = 2
NS = 16
STRIPE = N_PAD // NS
EPT = E_EDGES // (NC * NS)
CH = 40
NCHUNK = EPT // CH
BM = 2000
LAST = N_NODES - (NS - 1) * STRIPE


def _make_sc_agg(F, with_deg):
  mesh = plsc.VectorSubcoreMesh(core_axis_name="c", subcore_axis_name="s")
  outs = [jax.ShapeDtypeStruct((NC, N_NODES, F), jnp.float32)]
  scratch = [
      pltpu.VMEM((NCHUNK, CH), jnp.int32),
      pltpu.VMEM((NCHUNK, CH), jnp.int32),
      pltpu.VMEM((CH, F), jnp.float32),
      pltpu.VMEM((CH, F), jnp.float32),
      pltpu.VMEM((CH, F), jnp.float32),
      pltpu.VMEM((CH, F), jnp.float32),
      pltpu.VMEM_SHARED((N_PAD, F), jnp.float32),
      pltpu.SemaphoreType.DMA,
      pltpu.SemaphoreType.DMA,
      pltpu.SemaphoreType.DMA,
      pltpu.SemaphoreType.DMA,
      pltpu.SemaphoreType.DMA,
      pltpu.SemaphoreType.DMA,
      pltpu.SemaphoreType.DMA,
      pltpu.SemaphoreType.DMA,
  ]
  if with_deg:
    outs.append(jax.ShapeDtypeStruct((NC * N_NODES,), jnp.float32))
    scratch += [
        pltpu.VMEM((CH,), jnp.float32),
        pltpu.VMEM_SHARED((N_PAD,), jnp.float32),
        pltpu.VMEM((STRIPE,), jnp.float32),
        pltpu.SemaphoreType.DMA,
    ]

  def body(*refs):
    if with_deg:
      (table, src, dst, zrow, zvec, ones_h, out, degout,
       idx_s, idx_d, rb0, rb1, rb2, rb3, acc,
       sg0, sg1, sg2, sg3, ss0, ss1, ss2, ss3,
       ones_v, dega, dstage, sd) = refs
    else:
      (table, src, dst, zrow, out,
       idx_s, idx_d, rb0, rb1, rb2, rb3, acc,
       sg0, sg1, sg2, sg3, ss0, ss1, ss2, ss3) = refs
    RB = [rb0, rb1, rb2, rb3]
    SG = [sg0, sg1, sg2, sg3]
    SS = [ss0, ss1, ss2, ss3]
    s = lax.axis_index("s")
    c = lax.axis_index("c")
    wid = c * NS + s
    r0 = pl.multiple_of(s * STRIPE, 8)
    pltpu.sync_copy(src.at[wid], idx_s)
    pltpu.sync_copy(dst.at[wid], idx_d)
    pltpu.sync_copy(zrow, acc.at[pl.ds(r0, STRIPE), :])
    if with_deg:
      pltpu.sync_copy(zvec, dstage)
      pltpu.sync_copy(dstage, dega.at[pl.ds(r0, STRIPE)])
      pltpu.sync_copy(ones_h, ones_v)
    plsc.subcore_barrier()

    def gath(i, b):
      pltpu.async_copy(table.at[idx_s.at[i]], RB[b], SG[b])

    def wait_gath(b):
      pltpu.make_async_copy(table.at[idx_s.at[0]], RB[b], SG[b]).wait()

    def scat(i, b):
      pltpu.async_copy(RB[b], acc.at[idx_d.at[i]], SS[b], add=True)
      if with_deg:
        pltpu.async_copy(ones_v, dega.at[idx_d.at[i]], sd, add=True)

    def wait_scat(b):
      pltpu.make_async_copy(RB[b], acc.at[idx_d.at[0]], SS[b]).wait()

    gath(0, 0)
    gath(1, 1)
    wait_gath(0)
    scat(0, 0)
    gath(2, 2)
    wait_gath(1)
    scat(1, 1)
    gath(3, 3)

    def quad(j, carry):
      i = 4 * j + 2
      for t in range(4):
        b = (2 + t) % 4
        wait_gath(b)
        scat(i + t, b)
        wait_scat(t)
        gath(i + t + 2, t)
      return carry

    lax.fori_loop(0, (NCHUNK - 6) // 4, quad, 0)

    wait_gath(2)
    scat(NCHUNK - 4, 2)
    wait_scat(0)
    gath(NCHUNK - 2, 0)
    wait_gath(3)
    scat(NCHUNK - 3, 3)
    wait_scat(1)
    gath(NCHUNK - 1, 1)
    wait_gath(0)
    scat(NCHUNK - 2, 0)
    wait_gath(1)
    scat(NCHUNK - 1, 1)
    for b in (2, 3, 0, 1):
      wait_scat(b)
    if with_deg:
      pltpu.make_async_copy(src.at[wid], idx_s, sd).wait()
    plsc.subcore_barrier()

    @pl.when(s < NS - 1)
    def _():
      pltpu.sync_copy(acc.at[pl.ds(r0, STRIPE), :],
                      out.at[c, pl.ds(r0, STRIPE), :])
      if with_deg:
        d0 = pl.multiple_of(c * N_NODES + r0, 8)
        pltpu.sync_copy(dega.at[pl.ds(r0, STRIPE)], dstage)
        pltpu.sync_copy(dstage, degout.at[pl.ds(d0, STRIPE)])

    @pl.when(s == NS - 1)
    def _():
      pltpu.sync_copy(acc.at[pl.ds(r0, LAST), :],
                      out.at[c, pl.ds(r0, LAST), :])
      if with_deg:
        d0 = pl.multiple_of(c * N_NODES + r0, 8)
        pltpu.sync_copy(dega.at[pl.ds(r0, LAST)], dstage.at[pl.ds(0, LAST)])
        pltpu.sync_copy(dstage.at[pl.ds(0, LAST)], degout.at[pl.ds(d0, LAST)])

  return pl.kernel(body, mesh=mesh, out_type=outs, scratch_types=scratch,
                   compiler_params=pltpu.CompilerParams(
                       use_tc_tiling_on_sc=False))


_sc_agg_l1 = _make_sc_agg(H_HID, with_deg=True)
_sc_agg_l2 = _make_sc_agg(C_OUT, with_deg=False)


def _mm2_body(x_ref, wa_ref, wb_ref, oa_ref, ob_ref):
  xv = x_ref[...]
  oa_ref[...] = jnp.dot(xv, wa_ref[...], preferred_element_type=jnp.float32,
                        precision=lax.Precision.HIGHEST)
  ob_ref[...] = jnp.dot(xv, wb_ref[...], preferred_element_type=jnp.float32,
                        precision=lax.Precision.HIGHEST)


def _mm2(a, wa, wb):
  m, k = a.shape
  fa = wa.shape[1]
  fb = wb.shape[1]
  return pl.pallas_call(
      _mm2_body,
      grid=(m // BM,),
      in_specs=[
          pl.BlockSpec((BM, k), lambda i: (i, 0)),
          pl.BlockSpec((k, fa), lambda i: (0, 0)),
          pl.BlockSpec((k, fb), lambda i: (0, 0)),
      ],
      out_specs=[
          pl.BlockSpec((BM, fa), lambda i: (i, 0)),
          pl.BlockSpec((BM, fb), lambda i: (i, 0)),
      ],
      out_shape=[
          jax.ShapeDtypeStruct((m, fa), jnp.float32),
          jax.ShapeDtypeStruct((m, fb), jnp.float32),
      ],
  )(a, wa, wb)


def _mm_body(x_ref, w_ref, o_ref):
  o_ref[...] = jnp.dot(x_ref[...], w_ref[...], preferred_element_type=jnp.float32,
                       precision=lax.Precision.HIGHEST)


def _mm(a, w):
  m, k = a.shape
  f = w.shape[1]
  return pl.pallas_call(
      _mm_body,
      grid=(m // BM,),
      in_specs=[
          pl.BlockSpec((BM, k), lambda i: (i, 0)),
          pl.BlockSpec((k, f), lambda i: (0, 0)),
      ],
      out_specs=pl.BlockSpec((BM, f), lambda i: (i, 0)),
      out_shape=jax.ShapeDtypeStruct((m, f), jnp.float32),
  )(a, w)


def _l1mm_body(xs_ref, p_ref, dpt_ref, b_ref, wa_ref, wb_ref, oa_ref, ob_ref):
  d = jnp.maximum(dpt_ref[:, 0:1] + dpt_ref[:, 1:2], 1.0)
  agg = (p_ref[0] + p_ref[1]) / d
  h = jnp.maximum(xs_ref[...] + agg + b_ref[...], 0.0)
  oa_ref[...] = jnp.dot(h, wa_ref[...], preferred_element_type=jnp.float32,
                        precision=lax.Precision.HIGHEST)
  ob_ref[...] = jnp.dot(h, wb_ref[...], preferred_element_type=jnp.float32,
                        precision=lax.Precision.HIGHEST)


def _l1mm(xw_self, p, dpt, b, wa, wb):
  fa = wa.shape[1]
  fb = wb.shape[1]
  return pl.pallas_call(
      _l1mm_body,
      grid=(N_NODES // BM,),
      in_specs=[
          pl.BlockSpec((BM, H_HID), lambda i: (i, 0)),
          pl.BlockSpec((NC, BM, H_HID), lambda i: (0, i, 0)),
          pl.BlockSpec((BM, NC), lambda i: (i, 0)),
          pl.BlockSpec((1, H_HID), lambda i: (0, 0)),
          pl.BlockSpec((H_HID, fa), lambda i: (0, 0)),
          pl.BlockSpec((H_HID, fb), lambda i: (0, 0)),
      ],
      out_specs=[
          pl.BlockSpec((BM, fa), lambda i: (i, 0)),
          pl.BlockSpec((BM, fb), lambda i: (i, 0)),
      ],
      out_shape=[
          jax.ShapeDtypeStruct((N_NODES, fa), jnp.float32),
          jax.ShapeDtypeStruct((N_NODES, fb), jnp.float32),
      ],
  )(xw_self, p, dpt, b, wa, wb)


def _l2fin_body(hs_ref, q_ref, dpt_ref, b_ref, o_ref):
  d = jnp.maximum(dpt_ref[:, 0:1] + dpt_ref[:, 1:2], 1.0)
  z = hs_ref[...] + (q_ref[0] + q_ref[1]) / d + b_ref[...]
  m = jnp.max(z, axis=1, keepdims=True)
  e = z - m
  o_ref[...] = e - jnp.log(jnp.sum(jnp.exp(e), axis=1, keepdims=True))


def _l2fin(hw_self, q, dpt, b):
  return pl.pallas_call(
      _l2fin_body,
      grid=(N_NODES // BM,),
      in_specs=[
          pl.BlockSpec((BM, C_OUT), lambda i: (i, 0)),
          pl.BlockSpec((NC, BM, C_OUT), lambda i: (0, i, 0)),
          pl.BlockSpec((BM, NC), lambda i: (i, 0)),
          pl.BlockSpec((1, C_OUT), lambda i: (0, 0)),
      ],
      out_specs=pl.BlockSpec((BM, C_OUT), lambda i: (i, 0)),
      out_shape=jax.ShapeDtypeStruct((N_NODES, C_OUT), jnp.float32),
  )(hw_self, q, dpt, b)


def kernel(x, edge_index, W_self1, W_neigh1, b1, W_self2, W_neigh2, b2):
  src = edge_index[0].reshape(NC * NS, NCHUNK, CH)
  dst = edge_index[1].reshape(NC * NS, NCHUNK, CH)
  zrow1 = jnp.zeros((STRIPE, H_HID), jnp.float32)
  zrow2 = jnp.zeros((STRIPE, C_OUT), jnp.float32)
  zvec = jnp.zeros((STRIPE,), jnp.float32)
  ones_h = jnp.ones((CH,), jnp.float32)

  xw_neigh = _mm(x, W_neigh1)
  p, dp = _sc_agg_l1(xw_neigh, src, dst, zrow1, zvec, ones_h)
  xw_self = _mm(x, W_self1)
  dpt = dp.reshape(NC, N_NODES).T

  hw_neigh, hw_self = _l1mm(xw_self, p, dpt, b1.reshape(1, H_HID),
                            W_neigh2, W_self2)
  q, = _sc_agg_l2(hw_neigh, src, dst, zrow2)
  return _l2fin(hw_self, q, dpt, b2.reshape(1, C_OUT))


# speedup vs baseline: 1.1347x; 1.1347x over previous
import functools

import jax
import jax.numpy as jnp
from jax import lax
from jax.experimental import pallas as pl
from jax.experimental.pallas import tpu as pltpu
from jax.experimental.pallas import tpu_sc as plsc

N_NODES = 10000
N_PAD = 10240
E_EDGES = 320000
D_IN = 128
H_HID = 128
C_OUT = 64

NC = 2
NS = 16
STRIPE = N_PAD // NS
EPT = E_EDGES // (NC * NS)
CH = 80
NCHUNK = EPT // CH
BM = 2000
LAST = N_NODES - (NS - 1) * STRIPE


def _make_sc_agg(F, with_deg):
  mesh = plsc.VectorSubcoreMesh(core_axis_name="c", subcore_axis_name="s")
  outs = [jax.ShapeDtypeStruct((NC, N_NODES, F), jnp.float32)]
  scratch = [
      pltpu.VMEM((NCHUNK, CH), jnp.int32),
      pltpu.VMEM((NCHUNK, CH), jnp.int32),
      pltpu.VMEM((CH, F), jnp.float32),
      pltpu.VMEM((CH, F), jnp.float32),
      pltpu.VMEM_SHARED((N_PAD, F), jnp.float32),
      pltpu.SemaphoreType.DMA,
      pltpu.SemaphoreType.DMA,
      pltpu.SemaphoreType.DMA,
      pltpu.SemaphoreType.DMA,
  ]
  if with_deg:
    outs.append(jax.ShapeDtypeStruct((NC * N_NODES,), jnp.float32))
    scratch += [
        pltpu.VMEM((CH,), jnp.float32),
        pltpu.VMEM_SHARED((N_PAD,), jnp.float32),
        pltpu.VMEM((STRIPE,), jnp.float32),
        pltpu.SemaphoreType.DMA,
    ]

  def body(*refs):
    if with_deg:
      (table, src, dst, zrow, zvec, ones_h, out, degout,
       idx_s, idx_d, rb0, rb1, acc, sg0, sg1, ss0, ss1,
       ones_v, dega, dstage, sd) = refs
    else:
      (table, src, dst, zrow, out,
       idx_s, idx_d, rb0, rb1, acc, sg0, sg1, ss0, ss1) = refs
    RB = [rb0, rb1]
    SG = [sg0, sg1]
    SS = [ss0, ss1]
    s = lax.axis_index("s")
    c = lax.axis_index("c")
    wid = c * NS + s
    r0 = pl.multiple_of(s * STRIPE, 8)
    pltpu.sync_copy(src.at[wid], idx_s)
    pltpu.sync_copy(dst.at[wid], idx_d)
    pltpu.sync_copy(zrow, acc.at[pl.ds(r0, STRIPE), :])
    if with_deg:
      pltpu.sync_copy(zvec, dstage)
      pltpu.sync_copy(dstage, dega.at[pl.ds(r0, STRIPE)])
      pltpu.sync_copy(ones_h, ones_v)
    plsc.subcore_barrier()

    def gath(i, b):
      pltpu.async_copy(table.at[idx_s.at[i]], RB[b], SG[b])

    def wait_gath(b):
      pltpu.make_async_copy(table.at[idx_s.at[0]], RB[b], SG[b]).wait()

    def scat(i, b):
      pltpu.sync_copy(RB[b], acc.at[idx_d.at[i]], add=True)
      if with_deg:
        pltpu.async_copy(ones_v, dega.at[idx_d.at[i]], sd, add=True)

    gath(0, 0)

    def step(j, carry):
      i = 2 * j
      gath(i + 1, 1)
      wait_gath(0)
      scat(i, 0)
      gath(i + 2, 0)
      wait_gath(1)
      scat(i + 1, 1)
      return carry

    lax.fori_loop(0, (NCHUNK - 1) // 2, step, 0)
    wait_gath(0)
    scat(NCHUNK - 1, 0)
    if with_deg:
      pltpu.make_async_copy(src.at[wid], idx_s, sd).wait()
    plsc.subcore_barrier()

    @pl.when(s < NS - 1)
    def _():
      pltpu.sync_copy(acc.at[pl.ds(r0, STRIPE), :],
                      out.at[c, pl.ds(r0, STRIPE), :])
      if with_deg:
        d0 = pl.multiple_of(c * N_NODES + r0, 8)
        pltpu.sync_copy(dega.at[pl.ds(r0, STRIPE)], dstage)
        pltpu.sync_copy(dstage, degout.at[pl.ds(d0, STRIPE)])

    @pl.when(s == NS - 1)
    def _():
      pltpu.sync_copy(acc.at[pl.ds(r0, LAST), :],
                      out.at[c, pl.ds(r0, LAST), :])
      if with_deg:
        d0 = pl.multiple_of(c * N_NODES + r0, 8)
        pltpu.sync_copy(dega.at[pl.ds(r0, LAST)], dstage.at[pl.ds(0, LAST)])
        pltpu.sync_copy(dstage.at[pl.ds(0, LAST)], degout.at[pl.ds(d0, LAST)])

  return pl.kernel(body, mesh=mesh, out_type=outs, scratch_types=scratch,
                   compiler_params=pltpu.CompilerParams(
                       use_tc_tiling_on_sc=False))


_sc_agg_l1 = _make_sc_agg(H_HID, with_deg=True)
_sc_agg_l2 = _make_sc_agg(C_OUT, with_deg=False)


def _mm2_body(x_ref, wa_ref, wb_ref, oa_ref, ob_ref):
  xv = x_ref[...]
  oa_ref[...] = jnp.dot(xv, wa_ref[...], preferred_element_type=jnp.float32,
                        precision=lax.Precision.HIGHEST)
  ob_ref[...] = jnp.dot(xv, wb_ref[...], preferred_element_type=jnp.float32,
                        precision=lax.Precision.HIGHEST)


def _mm2(a, wa, wb):
  m, k = a.shape
  fa = wa.shape[1]
  fb = wb.shape[1]
  return pl.pallas_call(
      _mm2_body,
      grid=(m // BM,),
      in_specs=[
          pl.BlockSpec((BM, k), lambda i: (i, 0)),
          pl.BlockSpec((k, fa), lambda i: (0, 0)),
          pl.BlockSpec((k, fb), lambda i: (0, 0)),
      ],
      out_specs=[
          pl.BlockSpec((BM, fa), lambda i: (i, 0)),
          pl.BlockSpec((BM, fb), lambda i: (i, 0)),
      ],
      out_shape=[
          jax.ShapeDtypeStruct((m, fa), jnp.float32),
          jax.ShapeDtypeStruct((m, fb), jnp.float32),
      ],
  )(a, wa, wb)


def _mm_body(x_ref, w_ref, o_ref):
  o_ref[...] = jnp.dot(x_ref[...], w_ref[...], preferred_element_type=jnp.float32,
                       precision=lax.Precision.HIGHEST)


def _mm(a, w):
  m, k = a.shape
  f = w.shape[1]
  return pl.pallas_call(
      _mm_body,
      grid=(m // BM,),
      in_specs=[
          pl.BlockSpec((BM, k), lambda i: (i, 0)),
          pl.BlockSpec((k, f), lambda i: (0, 0)),
      ],
      out_specs=pl.BlockSpec((BM, f), lambda i: (i, 0)),
      out_shape=jax.ShapeDtypeStruct((m, f), jnp.float32),
  )(a, w)


def _l1mm_body(xs_ref, p_ref, dpt_ref, b_ref, wa_ref, wb_ref, oa_ref, ob_ref):
  d = jnp.maximum(dpt_ref[:, 0:1] + dpt_ref[:, 1:2], 1.0)
  agg = (p_ref[0] + p_ref[1]) / d
  h = jnp.maximum(xs_ref[...] + agg + b_ref[...], 0.0)
  oa_ref[...] = jnp.dot(h, wa_ref[...], preferred_element_type=jnp.float32,
                        precision=lax.Precision.HIGHEST)
  ob_ref[...] = jnp.dot(h, wb_ref[...], preferred_element_type=jnp.float32,
                        precision=lax.Precision.HIGHEST)


def _l1mm(xw_self, p, dpt, b, wa, wb):
  fa = wa.shape[1]
  fb = wb.shape[1]
  return pl.pallas_call(
      _l1mm_body,
      grid=(N_NODES // BM,),
      in_specs=[
          pl.BlockSpec((BM, H_HID), lambda i: (i, 0)),
          pl.BlockSpec((NC, BM, H_HID), lambda i: (0, i, 0)),
          pl.BlockSpec((BM, NC), lambda i: (i, 0)),
          pl.BlockSpec((1, H_HID), lambda i: (0, 0)),
          pl.BlockSpec((H_HID, fa), lambda i: (0, 0)),
          pl.BlockSpec((H_HID, fb), lambda i: (0, 0)),
      ],
      out_specs=[
          pl.BlockSpec((BM, fa), lambda i: (i, 0)),
          pl.BlockSpec((BM, fb), lambda i: (i, 0)),
      ],
      out_shape=[
          jax.ShapeDtypeStruct((N_NODES, fa), jnp.float32),
          jax.ShapeDtypeStruct((N_NODES, fb), jnp.float32),
      ],
  )(xw_self, p, dpt, b, wa, wb)


def _l2fin_body(hs_ref, q_ref, dpt_ref, b_ref, o_ref):
  d = jnp.maximum(dpt_ref[:, 0:1] + dpt_ref[:, 1:2], 1.0)
  z = hs_ref[...] + (q_ref[0] + q_ref[1]) / d + b_ref[...]
  m = jnp.max(z, axis=1, keepdims=True)
  e = z - m
  o_ref[...] = e - jnp.log(jnp.sum(jnp.exp(e), axis=1, keepdims=True))


def _l2fin(hw_self, q, dpt, b):
  return pl.pallas_call(
      _l2fin_body,
      grid=(N_NODES // BM,),
      in_specs=[
          pl.BlockSpec((BM, C_OUT), lambda i: (i, 0)),
          pl.BlockSpec((NC, BM, C_OUT), lambda i: (0, i, 0)),
          pl.BlockSpec((BM, NC), lambda i: (i, 0)),
          pl.BlockSpec((1, C_OUT), lambda i: (0, 0)),
      ],
      out_specs=pl.BlockSpec((BM, C_OUT), lambda i: (i, 0)),
      out_shape=jax.ShapeDtypeStruct((N_NODES, C_OUT), jnp.float32),
  )(hw_self, q, dpt, b)


def kernel(x, edge_index, W_self1, W_neigh1, b1, W_self2, W_neigh2, b2):
  src = edge_index[0].reshape(NC * NS, NCHUNK, CH)
  dst = edge_index[1].reshape(NC * NS, NCHUNK, CH)
  zrow1 = jnp.zeros((STRIPE, H_HID), jnp.float32)
  zrow2 = jnp.zeros((STRIPE, C_OUT), jnp.float32)
  zvec = jnp.zeros((STRIPE,), jnp.float32)
  ones_h = jnp.ones((CH,), jnp.float32)

  xw_neigh, xw_self = _mm2(x, W_neigh1, W_self1)
  p, dp = _sc_agg_l1(xw_neigh, src, dst, zrow1, zvec, ones_h)
  dpt = dp.reshape(NC, N_NODES).T

  hw_neigh, hw_self = _l1mm(xw_self, p, dpt, b1.reshape(1, H_HID),
                            W_neigh2, W_self2)
  q, = _sc_agg_l2(hw_neigh, src, dst, zrow2)
  return _l2fin(hw_self, q, dpt, b2.reshape(1, C_OUT))


# speedup vs baseline: 1.1865x; 1.0457x over previous
import functools

import jax
import jax.numpy as jnp
from jax import lax
from jax.experimental import pallas as pl
from jax.experimental.pallas import tpu as pltpu
from jax.experimental.pallas import tpu_sc as plsc

N_NODES = 10000
N_PAD = 10240
E_EDGES = 320000
D_IN = 128
H_HID = 128
C_OUT = 64

NC = 2
NS = 16
STRIPE = N_PAD // NS
EPT = E_EDGES // (NC * NS)
CH = 80
NCHUNK = EPT // CH
BM = 2000
LAST = N_NODES - (NS - 1) * STRIPE


def _make_sc_agg(F, with_deg, nbuf=2):
  mesh = plsc.VectorSubcoreMesh(core_axis_name="c", subcore_axis_name="s")
  outs = [jax.ShapeDtypeStruct((NC, N_NODES, F), jnp.float32)]
  scratch = [
      pltpu.VMEM((NCHUNK, CH), jnp.int32),
      pltpu.VMEM((NCHUNK, CH), jnp.int32),
  ]
  scratch += [pltpu.VMEM((CH, F), jnp.float32)] * nbuf
  scratch += [pltpu.VMEM_SHARED((N_PAD, F), jnp.float32)]
  scratch += [pltpu.SemaphoreType.DMA] * (2 * nbuf)
  if with_deg:
    outs.append(jax.ShapeDtypeStruct((NC * N_NODES,), jnp.float32))
    scratch += [
        pltpu.VMEM((CH,), jnp.float32),
        pltpu.VMEM_SHARED((N_PAD,), jnp.float32),
        pltpu.VMEM((STRIPE,), jnp.float32),
        pltpu.SemaphoreType.DMA,
    ]

  def body(*refs):
    if with_deg:
      (table, src, dst, zrow, zvec, ones_h, out, degout, idx_s, idx_d,
       *rest) = refs
      rest, (ones_v, dega, dstage, sd) = rest[:-4], rest[-4:]
    else:
      (table, src, dst, zrow, out, idx_s, idx_d, *rest) = refs
    RB = list(rest[:nbuf])
    acc = rest[nbuf]
    SG = list(rest[nbuf + 1:2 * nbuf + 1])
    SS = list(rest[2 * nbuf + 1:3 * nbuf + 1])
    s = lax.axis_index("s")
    c = lax.axis_index("c")
    wid = c * NS + s
    r0 = pl.multiple_of(s * STRIPE, 8)
    pltpu.sync_copy(src.at[wid], idx_s)
    pltpu.sync_copy(dst.at[wid], idx_d)
    pltpu.sync_copy(zrow, acc.at[pl.ds(r0, STRIPE), :])
    if with_deg:
      pltpu.sync_copy(zvec, dstage)
      pltpu.sync_copy(dstage, dega.at[pl.ds(r0, STRIPE)])
      pltpu.sync_copy(ones_h, ones_v)
    plsc.subcore_barrier()

    def gath(i, b):
      pltpu.async_copy(table.at[idx_s.at[i]], RB[b], SG[b])

    def wait_gath(b):
      pltpu.make_async_copy(table.at[idx_s.at[0]], RB[b], SG[b]).wait()

    def deg_scat(i):
      if with_deg:
        pltpu.async_copy(ones_v, dega.at[idx_d.at[i]], sd, add=True)

    def scat_sync(i, b):
      pltpu.sync_copy(RB[b], acc.at[idx_d.at[i]], add=True)
      deg_scat(i)

    def scat_async(i, b):
      pltpu.async_copy(RB[b], acc.at[idx_d.at[i]], SS[b], add=True)
      deg_scat(i)

    def wait_scat(b):
      pltpu.make_async_copy(RB[b], acc.at[idx_d.at[0]], SS[b]).wait()

    if nbuf == 2:
      gath(0, 0)

      def step(j, carry):
        i = 2 * j
        gath(i + 1, 1)
        wait_gath(0)
        scat_sync(i, 0)
        gath(i + 2, 0)
        wait_gath(1)
        scat_sync(i + 1, 1)
        return carry

      lax.fori_loop(0, (NCHUNK - 1) // 2, step, 0)
      wait_gath(0)
      scat_sync(NCHUNK - 1, 0)
    else:
      gath(0, 0)
      gath(1, 1)
      wait_gath(0)
      scat_async(0, 0)
      gath(2, 2)
      wait_gath(1)
      scat_async(1, 1)
      wait_scat(0)
      gath(3, 0)

      def tri(j, carry):
        i = 3 * j + 2
        for t in range(3):
          b = (2 + t) % 3
          wait_gath(b)
          scat_async(i + t, b)
          wait_scat((1 + t) % 3)
          gath(i + t + 2, (1 + t) % 3)
        return carry

      lax.fori_loop(0, (NCHUNK - 5) // 3, tri, 0)

      wait_gath(2)
      scat_async(NCHUNK - 3, 2)
      wait_scat(1)
      gath(NCHUNK - 1, 1)
      wait_gath(0)
      scat_async(NCHUNK - 2, 0)
      wait_gath(1)
      scat_async(NCHUNK - 1, 1)
      wait_scat(2)
      wait_scat(0)
      wait_scat(1)
    if with_deg:
      pltpu.make_async_copy(src.at[wid], idx_s, sd).wait()
    plsc.subcore_barrier()

    @pl.when(s < NS - 1)
    def _():
      pltpu.sync_copy(acc.at[pl.ds(r0, STRIPE), :],
                      out.at[c, pl.ds(r0, STRIPE), :])
      if with_deg:
        d0 = pl.multiple_of(c * N_NODES + r0, 8)
        pltpu.sync_copy(dega.at[pl.ds(r0, STRIPE)], dstage)
        pltpu.sync_copy(dstage, degout.at[pl.ds(d0, STRIPE)])

    @pl.when(s == NS - 1)
    def _():
      pltpu.sync_copy(acc.at[pl.ds(r0, LAST), :],
                      out.at[c, pl.ds(r0, LAST), :])
      if with_deg:
        d0 = pl.multiple_of(c * N_NODES + r0, 8)
        pltpu.sync_copy(dega.at[pl.ds(r0, LAST)], dstage.at[pl.ds(0, LAST)])
        pltpu.sync_copy(dstage.at[pl.ds(0, LAST)], degout.at[pl.ds(d0, LAST)])

  return pl.kernel(body, mesh=mesh, out_type=outs, scratch_types=scratch,
                   compiler_params=pltpu.CompilerParams(
                       use_tc_tiling_on_sc=False))


_sc_agg_l1 = _make_sc_agg(H_HID, with_deg=True, nbuf=2)
_sc_agg_l2 = _make_sc_agg(C_OUT, with_deg=False, nbuf=3)


def _mm2_body(x_ref, wa_ref, wb_ref, oa_ref, ob_ref):
  xv = x_ref[...]
  oa_ref[...] = jnp.dot(xv, wa_ref[...], preferred_element_type=jnp.float32,
                        precision=lax.Precision.HIGHEST)
  ob_ref[...] = jnp.dot(xv, wb_ref[...], preferred_element_type=jnp.float32,
                        precision=lax.Precision.HIGHEST)


def _mm2(a, wa, wb):
  m, k = a.shape
  fa = wa.shape[1]
  fb = wb.shape[1]
  return pl.pallas_call(
      _mm2_body,
      grid=(m // BM,),
      in_specs=[
          pl.BlockSpec((BM, k), lambda i: (i, 0)),
          pl.BlockSpec((k, fa), lambda i: (0, 0)),
          pl.BlockSpec((k, fb), lambda i: (0, 0)),
      ],
      out_specs=[
          pl.BlockSpec((BM, fa), lambda i: (i, 0)),
          pl.BlockSpec((BM, fb), lambda i: (i, 0)),
      ],
      out_shape=[
          jax.ShapeDtypeStruct((m, fa), jnp.float32),
          jax.ShapeDtypeStruct((m, fb), jnp.float32),
      ],
  )(a, wa, wb)


def _mm_body(x_ref, w_ref, o_ref):
  o_ref[...] = jnp.dot(x_ref[...], w_ref[...], preferred_element_type=jnp.float32,
                       precision=lax.Precision.HIGHEST)


def _mm(a, w):
  m, k = a.shape
  f = w.shape[1]
  return pl.pallas_call(
      _mm_body,
      grid=(m // BM,),
      in_specs=[
          pl.BlockSpec((BM, k), lambda i: (i, 0)),
          pl.BlockSpec((k, f), lambda i: (0, 0)),
      ],
      out_specs=pl.BlockSpec((BM, f), lambda i: (i, 0)),
      out_shape=jax.ShapeDtypeStruct((m, f), jnp.float32),
  )(a, w)


def _l1mm_body(xs_ref, p_ref, dpt_ref, b_ref, wa_ref, wb_ref, oa_ref, ob_ref):
  d = jnp.maximum(dpt_ref[:, 0:1] + dpt_ref[:, 1:2], 1.0)
  agg = (p_ref[0] + p_ref[1]) / d
  h = jnp.maximum(xs_ref[...] + agg + b_ref[...], 0.0)
  oa_ref[...] = jnp.dot(h, wa_ref[...], preferred_element_type=jnp.float32,
                        precision=lax.Precision.HIGHEST)
  ob_ref[...] = jnp.dot(h, wb_ref[...], preferred_element_type=jnp.float32,
                        precision=lax.Precision.HIGHEST)


def _l1mm(xw_self, p, dpt, b, wa, wb):
  fa = wa.shape[1]
  fb = wb.shape[1]
  return pl.pallas_call(
      _l1mm_body,
      grid=(N_NODES // BM,),
      in_specs=[
          pl.BlockSpec((BM, H_HID), lambda i: (i, 0)),
          pl.BlockSpec((NC, BM, H_HID), lambda i: (0, i, 0)),
          pl.BlockSpec((BM, NC), lambda i: (i, 0)),
          pl.BlockSpec((1, H_HID), lambda i: (0, 0)),
          pl.BlockSpec((H_HID, fa), lambda i: (0, 0)),
          pl.BlockSpec((H_HID, fb), lambda i: (0, 0)),
      ],
      out_specs=[
          pl.BlockSpec((BM, fa), lambda i: (i, 0)),
          pl.BlockSpec((BM, fb), lambda i: (i, 0)),
      ],
      out_shape=[
          jax.ShapeDtypeStruct((N_NODES, fa), jnp.float32),
          jax.ShapeDtypeStruct((N_NODES, fb), jnp.float32),
      ],
  )(xw_self, p, dpt, b, wa, wb)


def _l2fin_body(hs_ref, q_ref, dpt_ref, b_ref, o_ref):
  d = jnp.maximum(dpt_ref[:, 0:1] + dpt_ref[:, 1:2], 1.0)
  z = hs_ref[...] + (q_ref[0] + q_ref[1]) / d + b_ref[...]
  m = jnp.max(z, axis=1, keepdims=True)
  e = z - m
  o_ref[...] = e - jnp.log(jnp.sum(jnp.exp(e), axis=1, keepdims=True))


def _l2fin(hw_self, q, dpt, b):
  return pl.pallas_call(
      _l2fin_body,
      grid=(N_NODES // BM,),
      in_specs=[
          pl.BlockSpec((BM, C_OUT), lambda i: (i, 0)),
          pl.BlockSpec((NC, BM, C_OUT), lambda i: (0, i, 0)),
          pl.BlockSpec((BM, NC), lambda i: (i, 0)),
          pl.BlockSpec((1, C_OUT), lambda i: (0, 0)),
      ],
      out_specs=pl.BlockSpec((BM, C_OUT), lambda i: (i, 0)),
      out_shape=jax.ShapeDtypeStruct((N_NODES, C_OUT), jnp.float32),
  )(hw_self, q, dpt, b)


def kernel(x, edge_index, W_self1, W_neigh1, b1, W_self2, W_neigh2, b2):
  src = edge_index[0].reshape(NC * NS, NCHUNK, CH)
  dst = edge_index[1].reshape(NC * NS, NCHUNK, CH)
  zrow1 = jnp.zeros((STRIPE, H_HID), jnp.float32)
  zrow2 = jnp.zeros((STRIPE, C_OUT), jnp.float32)
  zvec = jnp.zeros((STRIPE,), jnp.float32)
  ones_h = jnp.ones((CH,), jnp.float32)

  xw_neigh, xw_self = _mm2(x, W_neigh1, W_self1)
  p, dp = _sc_agg_l1(xw_neigh, src, dst, zrow1, zvec, ones_h)
  dpt = dp.reshape(NC, N_NODES).T

  hw_neigh, hw_self = _l1mm(xw_self, p, dpt, b1.reshape(1, H_HID),
                            W_neigh2, W_self2)
  q, = _sc_agg_l2(hw_neigh, src, dst, zrow2)
  return _l2fin(hw_self, q, dpt, b2.reshape(1, C_OUT))


# speedup vs baseline: 1.2684x; 1.0690x over previous
import functools

import jax
import jax.numpy as jnp
from jax import lax
from jax.experimental import pallas as pl
from jax.experimental.pallas import tpu as pltpu
from jax.experimental.pallas import tpu_sc as plsc

N_NODES = 10000
N_PAD = 10240
E_EDGES = 320000
D_IN = 128
H_HID = 128
C_OUT = 64

NC = 2
NS = 16
STRIPE = N_PAD // NS
EPT = E_EDGES // (NC * NS)
CH = 80
NCHUNK = EPT // CH
BM = 2000
LAST = N_NODES - (NS - 1) * STRIPE


def _make_sc_agg(F, with_deg, nbuf=2, src_ring=False):
  mesh = plsc.VectorSubcoreMesh(core_axis_name="c", subcore_axis_name="s")
  outs = [jax.ShapeDtypeStruct((NC, N_NODES, F), jnp.float32)]
  assert not src_ring or nbuf == 3
  scratch = [
      pltpu.VMEM((3, CH) if src_ring else (NCHUNK, CH), jnp.int32),
      pltpu.VMEM((NCHUNK, CH), jnp.int32),
  ]
  scratch += [pltpu.VMEM((CH, F), jnp.float32)] * nbuf
  scratch += [pltpu.VMEM_SHARED((N_PAD, F), jnp.float32)]
  scratch += [pltpu.SemaphoreType.DMA] * (2 * nbuf)
  if src_ring:
    scratch += [pltpu.SemaphoreType.DMA] * 3
  if with_deg:
    outs.append(jax.ShapeDtypeStruct((NC * N_NODES,), jnp.float32))
    scratch += [
        pltpu.VMEM((CH,), jnp.float32),
        pltpu.VMEM_SHARED((N_PAD,), jnp.float32),
        pltpu.VMEM((STRIPE,), jnp.float32),
        pltpu.SemaphoreType.DMA,
    ]

  def body(*refs):
    if with_deg:
      (table, src, dst, zrow, zvec, ones_h, out, degout, idx_s, idx_d,
       *rest) = refs
      rest, (ones_v, dega, dstage, sd) = rest[:-4], rest[-4:]
    else:
      (table, src, dst, zrow, out, idx_s, idx_d, *rest) = refs
    RB = list(rest[:nbuf])
    acc = rest[nbuf]
    SG = list(rest[nbuf + 1:2 * nbuf + 1])
    SS = list(rest[2 * nbuf + 1:3 * nbuf + 1])
    SI = list(rest[3 * nbuf + 1:3 * nbuf + 4]) if src_ring else None
    s = lax.axis_index("s")
    c = lax.axis_index("c")
    wid = c * NS + s
    r0 = pl.multiple_of(s * STRIPE, 8)
    if not src_ring:
      pltpu.sync_copy(src.at[wid], idx_s)
    pltpu.sync_copy(dst.at[wid], idx_d)
    pltpu.sync_copy(zrow, acc.at[pl.ds(r0, STRIPE), :])
    if with_deg:
      pltpu.sync_copy(zvec, dstage)
      pltpu.sync_copy(dstage, dega.at[pl.ds(r0, STRIPE)])
      pltpu.sync_copy(ones_h, ones_v)
    plsc.subcore_barrier()

    def src_idx(i, b):
      return idx_s.at[b] if src_ring else idx_s.at[i]

    def gath(i, b):
      pltpu.async_copy(table.at[src_idx(i, b)], RB[b], SG[b])

    def wait_gath(b):
      pltpu.make_async_copy(table.at[src_idx(0, b)], RB[b], SG[b]).wait()

    def idx_load(i, slot):
      pltpu.async_copy(src.at[wid, i], idx_s.at[slot], SI[slot])

    def wait_idx(slot):
      pltpu.make_async_copy(src.at[wid, 0], idx_s.at[slot], SI[slot]).wait()

    def deg_scat(i):
      if with_deg:
        pltpu.async_copy(ones_v, dega.at[idx_d.at[i]], sd, add=True)

    def scat_sync(i, b):
      pltpu.sync_copy(RB[b], acc.at[idx_d.at[i]], add=True)
      deg_scat(i)

    def scat_async(i, b):
      pltpu.async_copy(RB[b], acc.at[idx_d.at[i]], SS[b], add=True)
      deg_scat(i)

    def wait_scat(b):
      pltpu.make_async_copy(RB[b], acc.at[idx_d.at[0]], SS[b]).wait()

    if nbuf == 2:
      gath(0, 0)

      def step(j, carry):
        i = 2 * j
        gath(i + 1, 1)
        wait_gath(0)
        scat_sync(i, 0)
        gath(i + 2, 0)
        wait_gath(1)
        scat_sync(i + 1, 1)
        return carry

      lax.fori_loop(0, (NCHUNK - 1) // 2, step, 0)
      wait_gath(0)
      scat_sync(NCHUNK - 1, 0)
    else:
      if src_ring:
        idx_load(0, 0)
        idx_load(1, 1)
        idx_load(2, 2)
        wait_idx(0)
        wait_idx(1)
        wait_idx(2)
      gath(0, 0)
      gath(1, 1)
      wait_gath(0)
      if src_ring:
        idx_load(3, 0)
      scat_async(0, 0)
      gath(2, 2)
      wait_gath(1)
      if src_ring:
        idx_load(4, 1)
      scat_async(1, 1)
      wait_scat(0)
      if src_ring:
        wait_idx(0)
      gath(3, 0)

      def tri(j, carry):
        i = 3 * j + 2
        for t in range(3):
          b = (2 + t) % 3
          wait_gath(b)
          if src_ring:
            idx_load(i + t + 3, b)
          scat_async(i + t, b)
          wait_scat((1 + t) % 3)
          if src_ring:
            wait_idx((1 + t) % 3)
          gath(i + t + 2, (1 + t) % 3)
        return carry

      lax.fori_loop(0, (NCHUNK - 5) // 3, tri, 0)

      wait_gath(2)
      scat_async(NCHUNK - 3, 2)
      wait_scat(1)
      if src_ring:
        wait_idx(1)
      gath(NCHUNK - 1, 1)
      wait_gath(0)
      scat_async(NCHUNK - 2, 0)
      wait_gath(1)
      scat_async(NCHUNK - 1, 1)
      wait_scat(2)
      wait_scat(0)
      wait_scat(1)
    if with_deg:
      pltpu.make_async_copy(dst.at[wid], idx_d, sd).wait()
    plsc.subcore_barrier()

    @pl.when(s < NS - 1)
    def _():
      pltpu.sync_copy(acc.at[pl.ds(r0, STRIPE), :],
                      out.at[c, pl.ds(r0, STRIPE), :])
      if with_deg:
        d0 = pl.multiple_of(c * N_NODES + r0, 8)
        pltpu.sync_copy(dega.at[pl.ds(r0, STRIPE)], dstage)
        pltpu.sync_copy(dstage, degout.at[pl.ds(d0, STRIPE)])

    @pl.when(s == NS - 1)
    def _():
      pltpu.sync_copy(acc.at[pl.ds(r0, LAST), :],
                      out.at[c, pl.ds(r0, LAST), :])
      if with_deg:
        d0 = pl.multiple_of(c * N_NODES + r0, 8)
        pltpu.sync_copy(dega.at[pl.ds(r0, LAST)], dstage.at[pl.ds(0, LAST)])
        pltpu.sync_copy(dstage.at[pl.ds(0, LAST)], degout.at[pl.ds(d0, LAST)])

  return pl.kernel(body, mesh=mesh, out_type=outs, scratch_types=scratch,
                   compiler_params=pltpu.CompilerParams(
                       use_tc_tiling_on_sc=False))


_sc_agg_l1 = _make_sc_agg(H_HID, with_deg=True, nbuf=3, src_ring=True)
_sc_agg_l2 = _make_sc_agg(C_OUT, with_deg=False, nbuf=3)


def _mm2_body(x_ref, wa_ref, wb_ref, oa_ref, ob_ref):
  xv = x_ref[...]
  oa_ref[...] = jnp.dot(xv, wa_ref[...], preferred_element_type=jnp.float32,
                        precision=lax.Precision.HIGHEST)
  ob_ref[...] = jnp.dot(xv, wb_ref[...], preferred_element_type=jnp.float32,
                        precision=lax.Precision.HIGHEST)


def _mm2(a, wa, wb):
  m, k = a.shape
  fa = wa.shape[1]
  fb = wb.shape[1]
  return pl.pallas_call(
      _mm2_body,
      grid=(m // BM,),
      in_specs=[
          pl.BlockSpec((BM, k), lambda i: (i, 0)),
          pl.BlockSpec((k, fa), lambda i: (0, 0)),
          pl.BlockSpec((k, fb), lambda i: (0, 0)),
      ],
      out_specs=[
          pl.BlockSpec((BM, fa), lambda i: (i, 0)),
          pl.BlockSpec((BM, fb), lambda i: (i, 0)),
      ],
      out_shape=[
          jax.ShapeDtypeStruct((m, fa), jnp.float32),
          jax.ShapeDtypeStruct((m, fb), jnp.float32),
      ],
  )(a, wa, wb)


def _mm_body(x_ref, w_ref, o_ref):
  o_ref[...] = jnp.dot(x_ref[...], w_ref[...], preferred_element_type=jnp.float32,
                       precision=lax.Precision.HIGHEST)


def _mm(a, w):
  m, k = a.shape
  f = w.shape[1]
  return pl.pallas_call(
      _mm_body,
      grid=(m // BM,),
      in_specs=[
          pl.BlockSpec((BM, k), lambda i: (i, 0)),
          pl.BlockSpec((k, f), lambda i: (0, 0)),
      ],
      out_specs=pl.BlockSpec((BM, f), lambda i: (i, 0)),
      out_shape=jax.ShapeDtypeStruct((m, f), jnp.float32),
  )(a, w)


def _l1mm_body(xs_ref, p_ref, dpt_ref, b_ref, wa_ref, wb_ref, oa_ref, ob_ref):
  d = jnp.maximum(dpt_ref[:, 0:1] + dpt_ref[:, 1:2], 1.0)
  agg = (p_ref[0] + p_ref[1]) / d
  h = jnp.maximum(xs_ref[...] + agg + b_ref[...], 0.0)
  oa_ref[...] = jnp.dot(h, wa_ref[...], preferred_element_type=jnp.float32,
                        precision=lax.Precision.HIGHEST)
  ob_ref[...] = jnp.dot(h, wb_ref[...], preferred_element_type=jnp.float32,
                        precision=lax.Precision.HIGHEST)


def _l1mm(xw_self, p, dpt, b, wa, wb):
  fa = wa.shape[1]
  fb = wb.shape[1]
  return pl.pallas_call(
      _l1mm_body,
      grid=(N_NODES // BM,),
      in_specs=[
          pl.BlockSpec((BM, H_HID), lambda i: (i, 0)),
          pl.BlockSpec((NC, BM, H_HID), lambda i: (0, i, 0)),
          pl.BlockSpec((BM, NC), lambda i: (i, 0)),
          pl.BlockSpec((1, H_HID), lambda i: (0, 0)),
          pl.BlockSpec((H_HID, fa), lambda i: (0, 0)),
          pl.BlockSpec((H_HID, fb), lambda i: (0, 0)),
      ],
      out_specs=[
          pl.BlockSpec((BM, fa), lambda i: (i, 0)),
          pl.BlockSpec((BM, fb), lambda i: (i, 0)),
      ],
      out_shape=[
          jax.ShapeDtypeStruct((N_NODES, fa), jnp.float32),
          jax.ShapeDtypeStruct((N_NODES, fb), jnp.float32),
      ],
  )(xw_self, p, dpt, b, wa, wb)


def _l2fin_body(hs_ref, q_ref, dpt_ref, b_ref, o_ref):
  d = jnp.maximum(dpt_ref[:, 0:1] + dpt_ref[:, 1:2], 1.0)
  z = hs_ref[...] + (q_ref[0] + q_ref[1]) / d + b_ref[...]
  m = jnp.max(z, axis=1, keepdims=True)
  e = z - m
  o_ref[...] = e - jnp.log(jnp.sum(jnp.exp(e), axis=1, keepdims=True))


def _l2fin(hw_self, q, dpt, b):
  return pl.pallas_call(
      _l2fin_body,
      grid=(N_NODES // BM,),
      in_specs=[
          pl.BlockSpec((BM, C_OUT), lambda i: (i, 0)),
          pl.BlockSpec((NC, BM, C_OUT), lambda i: (0, i, 0)),
          pl.BlockSpec((BM, NC), lambda i: (i, 0)),
          pl.BlockSpec((1, C_OUT), lambda i: (0, 0)),
      ],
      out_specs=pl.BlockSpec((BM, C_OUT), lambda i: (i, 0)),
      out_shape=jax.ShapeDtypeStruct((N_NODES, C_OUT), jnp.float32),
  )(hw_self, q, dpt, b)


def kernel(x, edge_index, W_self1, W_neigh1, b1, W_self2, W_neigh2, b2):
  src = edge_index[0].reshape(NC * NS, NCHUNK, CH)
  dst = edge_index[1].reshape(NC * NS, NCHUNK, CH)
  zrow1 = jnp.zeros((STRIPE, H_HID), jnp.float32)
  zrow2 = jnp.zeros((STRIPE, C_OUT), jnp.float32)
  zvec = jnp.zeros((STRIPE,), jnp.float32)
  ones_h = jnp.ones((CH,), jnp.float32)

  xw_neigh, xw_self = _mm2(x, W_neigh1, W_self1)
  p, dp = _sc_agg_l1(xw_neigh, src, dst, zrow1, zvec, ones_h)
  dpt = dp.reshape(NC, N_NODES).T

  hw_neigh, hw_self = _l1mm(xw_self, p, dpt, b1.reshape(1, H_HID),
                            W_neigh2, W_self2)
  q, = _sc_agg_l2(hw_neigh, src, dst, zrow2)
  return _l2fin(hw_self, q, dpt, b2.reshape(1, C_OUT))
